# baseline (speedup 1.0000x reference)
import functools
import jax
import jax.numpy as jnp
from jax import lax
from jax.experimental import pallas as pl
from jax.experimental.pallas import tpu as pltpu
from jax.experimental.pallas import tpu_sc as plsc

L = 16
NC, NS = 2, 16
NW = NC * NS
CH = 4
DVEC = 12


def kernel(x, xemb, yemb):
    B, H, W, D = x.shape
    row = W * D
    nrows = B * H
    rpw = nrows // NW
    nch = rpw // CH

    x2 = x.reshape(nrows, row)
    xe_flat = xemb.reshape(H * D)
    ye_flat = yemb.reshape(row)

    mesh = plsc.VectorSubcoreMesh(
        core_axis_name="c", subcore_axis_name="s", num_cores=NC, num_subcores=NS
    )

    @functools.partial(
        pl.kernel,
        mesh=mesh,
        out_type=jax.ShapeDtypeStruct((nrows, row), jnp.float32),
        scratch_types=[
            pltpu.VMEM((H * D,), jnp.float32),
            pltpu.VMEM((row,), jnp.float32),
            pltpu.VMEM((2, CH, row), jnp.float32),
            pltpu.VMEM((2, CH, row), jnp.float32),
            pltpu.SemaphoreType.DMA((2,)),
            pltpu.SemaphoreType.DMA((2,)),
        ],
    )
    def _sc(x_hbm, xe_hbm, ye_hbm, o_hbm, xe_v, ye_v, ibuf, obuf, isem, osem):
        wid = lax.axis_index("s") * NC + lax.axis_index("c")
        base = wid * rpw

        pltpu.sync_copy(xe_hbm, xe_v)
        pltpu.sync_copy(ye_hbm, ye_v)

        def start_in(c, slot):
            pltpu.make_async_copy(
                x_hbm.at[pl.ds(base + c * CH, CH)], ibuf.at[slot], isem.at[slot]
            ).start()

        start_in(0, 0)
        start_in(1, 1)

        @pl.loop(0, nch)
        def _chunk(c):
            slot = lax.rem(c, 2)
            pltpu.make_async_copy(
                x_hbm.at[pl.ds(base + c * CH, CH)], ibuf.at[slot], isem.at[slot]
            ).wait()

            @pl.when(c >= 2)
            def _():
                pltpu.make_async_copy(
                    obuf.at[slot],
                    o_hbm.at[pl.ds(base + (c - 2) * CH, CH)],
                    osem.at[slot],
                ).wait()

            h0 = lax.rem(c * CH, H)
            for rloc in range(CH):
                xbase = (h0 + rloc) * D
                xv = tuple(
                    xe_v[pl.ds(xbase + j * L, L)] for j in range(DVEC)
                )

                @plsc.parallel_loop(0, W, unroll=2, carry=xv)
                def _w(w, xv):
                    for j in range(DVEC):
                        col = w * D + j * L
                        obuf[slot, rloc, pl.ds(col, L)] = (
                            ibuf[slot, rloc, pl.ds(col, L)]
                            + ye_v[pl.ds(col, L)]
                            + xv[j]
                        )
                    return xv

            pltpu.make_async_copy(
                obuf.at[slot], o_hbm.at[pl.ds(base + c * CH, CH)], osem.at[slot]
            ).start()

            @pl.when(c + 2 < nch)
            def _():
                start_in(c + 2, slot)

        for slot in range(2):
            pltpu.make_async_copy(
                obuf.at[slot], o_hbm.at[pl.ds(base, CH)], osem.at[slot]
            ).wait()

    out = _sc(x2, xe_flat, ye_flat)
    return out.reshape(B, H, W, D)


# speedup vs baseline: 1.0096x; 1.0096x over previous
import functools
import jax
import jax.numpy as jnp
from jax import lax
from jax.experimental import pallas as pl
from jax.experimental.pallas import tpu as pltpu
from jax.experimental.pallas import tpu_sc as plsc

L = 16
NC, NS = 2, 16
NW = NC * NS
CH = 4
DVEC = 12


def kernel(x, xemb, yemb):
    B, H, W, D = x.shape
    row = W * D
    nrows = B * H
    rpw = nrows // NW
    nch = rpw // CH

    x2 = x.reshape(nrows, row)
    xe_flat = xemb.reshape(H * D)
    ye_flat = yemb.reshape(row)

    mesh = plsc.VectorSubcoreMesh(
        core_axis_name="c", subcore_axis_name="s", num_cores=NC, num_subcores=NS
    )

    @functools.partial(
        pl.kernel,
        mesh=mesh,
        out_type=jax.ShapeDtypeStruct((nrows, row), jnp.float32),
        scratch_types=[
            pltpu.VMEM((H * D,), jnp.float32),
            pltpu.VMEM((row,), jnp.float32),
            pltpu.VMEM((2, CH, row), jnp.float32),
            pltpu.VMEM((2, CH, row), jnp.float32),
            pltpu.SemaphoreType.DMA((2,)),
            pltpu.SemaphoreType.DMA((2,)),
        ],
    )
    def _sc(x_hbm, xe_hbm, ye_hbm, o_hbm, xe_v, ye_v, ibuf, obuf, isem, osem):
        wid = lax.axis_index("s") * NC + lax.axis_index("c")
        base = wid * rpw

        pltpu.sync_copy(xe_hbm, xe_v)
        pltpu.sync_copy(ye_hbm, ye_v)

        def start_in(c, slot):
            pltpu.make_async_copy(
                x_hbm.at[pl.ds(base + c * CH, CH)], ibuf.at[slot], isem.at[slot]
            ).start()

        start_in(0, 0)
        start_in(1, 1)

        @pl.loop(0, nch)
        def _chunk(c):
            slot = lax.rem(c, 2)
            pltpu.make_async_copy(
                x_hbm.at[pl.ds(base + c * CH, CH)], ibuf.at[slot], isem.at[slot]
            ).wait()

            @pl.when(c >= 2)
            def _():
                pltpu.make_async_copy(
                    obuf.at[slot],
                    o_hbm.at[pl.ds(base + (c - 2) * CH, CH)],
                    osem.at[slot],
                ).wait()

            h0 = lax.rem(c * CH, H)
            for j in range(DVEC):
                xv = tuple(
                    xe_v[pl.ds((h0 + r) * D + j * L, L)] for r in range(CH)
                )

                @plsc.parallel_loop(0, W, unroll=2, carry=xv)
                def _w(w, xv):
                    col = w * D + j * L
                    ye = ye_v[pl.ds(col, L)]
                    for r in range(CH):
                        obuf[slot, r, pl.ds(col, L)] = (
                            ibuf[slot, r, pl.ds(col, L)] + ye + xv[r]
                        )
                    return xv

            pltpu.make_async_copy(
                obuf.at[slot], o_hbm.at[pl.ds(base + c * CH, CH)], osem.at[slot]
            ).start()

            @pl.when(c + 2 < nch)
            def _():
                start_in(c + 2, slot)

        for slot in range(2):
            pltpu.make_async_copy(
                obuf.at[slot], o_hbm.at[pl.ds(base, CH)], osem.at[slot]
            ).wait()

    out = _sc(x2, xe_flat, ye_flat)
    return out.reshape(B, H, W, D)


# speedup vs baseline: 1.4583x; 1.4444x over previous
import functools
import jax
import jax.numpy as jnp
from jax import lax
from jax.experimental import pallas as pl
from jax.experimental.pallas import tpu as pltpu
from jax.experimental.pallas import tpu_sc as plsc

L = 16
NC, NS = 2, 16
NW = NC * NS
NBUF = 4


def kernel(x, xemb, yemb):
    B, H, W, D = x.shape
    row = W * D
    nvec = row // L
    bpw = B // NW

    mesh = plsc.VectorSubcoreMesh(
        core_axis_name="c", subcore_axis_name="s", num_cores=NC, num_subcores=NS
    )

    @functools.partial(
        pl.kernel,
        mesh=mesh,
        out_type=jax.ShapeDtypeStruct((B, H, W, D), jnp.float32),
        scratch_types=[
            pltpu.VMEM((H, D), jnp.float32),
            pltpu.VMEM((W, D), jnp.float32),
            pltpu.VMEM((W, D), jnp.float32),
            pltpu.VMEM((NBUF, W, D), jnp.float32),
            pltpu.VMEM((NBUF, W, D), jnp.float32),
            pltpu.SemaphoreType.DMA((NBUF,)),
            pltpu.SemaphoreType.DMA((NBUF,)),
        ],
    )
    def _sc(x_hbm, xe_hbm, ye_hbm, o_hbm, xe_v, ye_v, pos_v, ibuf, obuf,
            isem, osem):
        wid = lax.axis_index("s") * NC + lax.axis_index("c")
        b0 = wid * bpw

        pltpu.sync_copy(xe_hbm, xe_v)
        pltpu.sync_copy(ye_hbm, ye_v)

        def start_in(t, slot):
            h = t // bpw
            b = b0 + lax.rem(t, bpw)
            pltpu.make_async_copy(
                x_hbm.at[b, h], ibuf.at[slot], isem.at[slot]
            ).start()

        for j in range(NBUF):
            start_in(j, j)

        @pl.loop(0, H)
        def _h(h):
            xrow = tuple(xe_v[h, pl.ds(j * L, L)] for j in range(D // L))

            @plsc.parallel_loop(0, W, unroll=2, carry=xrow)
            def _pw(w, xrow):
                for j in range(D // L):
                    pos_v[w, pl.ds(j * L, L)] = (
                        ye_v[w, pl.ds(j * L, L)] + xrow[j]
                    )
                return xrow

            @pl.loop(0, bpw)
            def _b(bl):
                t = h * bpw + bl
                slot = lax.rem(t, NBUF)
                pltpu.make_async_copy(
                    x_hbm.at[b0 + bl, h], ibuf.at[slot], isem.at[slot]
                ).wait()

                @pl.when(t >= NBUF)
                def _():
                    tp = t - NBUF
                    hp = tp // bpw
                    bp = b0 + lax.rem(tp, bpw)
                    pltpu.make_async_copy(
                        obuf.at[slot], o_hbm.at[bp, hp], osem.at[slot]
                    ).wait()

                @plsc.parallel_loop(0, W, unroll=2)
                def _v(w):
                    for j in range(D // L):
                        sl = pl.ds(j * L, L)
                        obuf[slot, w, sl] = ibuf[slot, w, sl] + pos_v[w, sl]

                pltpu.make_async_copy(
                    obuf.at[slot], o_hbm.at[b0 + bl, h], osem.at[slot]
                ).start()

                @pl.when(t + NBUF < H * bpw)
                def _():
                    start_in(t + NBUF, slot)

        for j in range(NBUF):
            pltpu.make_async_copy(
                obuf.at[j], o_hbm.at[b0, 0], osem.at[j]
            ).wait()

    return _sc(x, xemb, yemb)
